# Initial kernel scaffold; baseline (speedup 1.0000x reference)
import jax
import jax.numpy as jnp
from jax.experimental import pallas as pl


def kernel(spk_in, W1, W2, W3):
    raise NotImplementedError("write your pallas kernel here")


# speedup vs baseline: 13.9808x; 13.9808x over previous
import jax
import jax.numpy as jnp
from jax import lax
from jax.experimental import pallas as pl


def _pad2d(x, p):
    C, H, W = x.shape
    zc = jnp.zeros((C, H, p), x.dtype)
    x = jnp.concatenate([zc, x, zc], axis=2)
    zr = jnp.zeros((C, p, W + 2 * p), x.dtype)
    return jnp.concatenate([zr, x, zr], axis=1)


def _spiking_layer(x, Wf, K, pad, thr):
    C, H, W = x.shape
    O = Wf.shape[0]
    HW = H * W
    xp = _pad2d(x, pad)
    ip = (xp > 0).astype(jnp.float32)
    cols = []
    for kh in range(K):
        for kw in range(K):
            s = xp[:, kh:kh + H, kw:kw + W].reshape(C, HW)
            si = ip[:, kh:kh + H, kw:kw + W].reshape(C, HW)
            cols.append(jnp.concatenate([si, s], axis=1))
    X = jnp.concatenate(cols, axis=0)
    P = jnp.dot(Wf, X, preferred_element_type=jnp.float32)
    pot = P[:, :HW]
    tnum = P[:, HW:]
    m = jnp.max(pot, axis=0, keepdims=True)
    cidx = lax.broadcasted_iota(jnp.int32, (O, HW), 0)
    amax = jnp.min(jnp.where(pot == m, cidx, O), axis=0, keepdims=True)
    mask = (cidx == amax) & (m > thr)
    out = jnp.where(mask, tnum / jnp.maximum(pot, 1e-6), 0.0)
    return out


def _pool2x2_flat(x, O, H, W):
    W2, H2 = W // 2, H // 2
    x = x.reshape(O * H2, 2 * W)
    y = jnp.maximum(x[:, :W], x[:, W:])
    r = lax.broadcasted_iota(jnp.int32, (W, W2), 0)
    c = lax.broadcasted_iota(jnp.int32, (W, W2), 1)
    s_even = (r == 2 * c).astype(jnp.float32)
    s_odd = (r == 2 * c + 1).astype(jnp.float32)
    z = jnp.maximum(
        jnp.dot(y, s_even, preferred_element_type=jnp.float32),
        jnp.dot(y, s_odd, preferred_element_type=jnp.float32),
    )
    return z.reshape(O, H2, W2)


def _csnn_kernel(spk_ref, w1_ref, w2_ref, w3_ref, out_ref):
    x = spk_ref[...]
    x = _spiking_layer(x, w1_ref[...], 5, 2, 2.4)
    x = _pool2x2_flat(x, 30, 128, 128)
    x = _spiking_layer(x, w2_ref[...], 3, 1, 1.0)
    x = _pool2x2_flat(x, 100, 64, 64)
    x = _spiking_layer(x, w3_ref[...], 3, 1, 1.0)
    out_ref[...] = x


def kernel(spk_in, W1, W2, W3):
    w1f = jnp.transpose(W1, (0, 2, 3, 1)).reshape(30, 5 * 5 * 2)
    w2f = jnp.transpose(W2, (0, 2, 3, 1)).reshape(100, 3 * 3 * 30)
    w3f = jnp.transpose(W3, (0, 2, 3, 1)).reshape(200, 3 * 3 * 100)
    out = pl.pallas_call(
        _csnn_kernel,
        out_shape=jax.ShapeDtypeStruct((200, 32 * 32), jnp.float32),
    )(spk_in, w1f, w2f, w3f)
    return out.reshape(200, 32, 32)
